# baseline (speedup 1.0000x reference)
import functools

import jax
import jax.numpy as jnp
from jax import lax
from jax.experimental import pallas as pl
from jax.experimental.pallas import tpu as pltpu
from jax.experimental.pallas import tpu_sc as plsc

B = 4096
C = 1000
V = 50
D = 128
ROW = V * D
VPAD = 64
NW = 32
EPW = B // NW
CH = 8
NCHUNK = EPW // CH
NK = D // 16
EPS2 = 1e-16


def _pnorm_body(x_ref, o_ref):
    x = x_ref[...]
    n2 = jnp.sum(x * x, axis=-1, keepdims=True)
    o_ref[...] = x * lax.rsqrt(jnp.maximum(n2, EPS2))


def _gnorm_body(x_ref, o_ref):
    x = x_ref[...]
    n2 = jnp.sum(x * x, axis=-1, keepdims=True)
    o_ref[...] = x * lax.rsqrt(jnp.maximum(n2, EPS2))


def _sc_body(gemd, cid, ptab, out, idx_v, b_v, rows_v, dbuf, obuf, sem):
    wid = lax.axis_index("s") * 2 + lax.axis_index("c")
    base = wid * EPW
    iota = lax.iota(jnp.int32, 16)

    def chunk_body(c, carry):
        eb = base + c * CH
        pltpu.sync_copy(cid.at[pl.ds(eb, CH)], idx_v)
        pltpu.sync_copy(gemd.at[pl.ds(eb, CH)], b_v)
        pltpu.async_copy(ptab.at[idx_v], rows_v, sem).wait()

        def elem_body(e, ecarry):
            bks = [b_v[e, pl.ds(k * 16, 16)] for k in range(NK)]
            for g in range(4):
                nj = 16 if g < 3 else V - 48
                for j in range(nj):
                    v = g * 16 + j
                    av = rows_v[e, pl.ds(v * D, 16)]
                    accd = av * bks[0]
                    for k in range(1, NK):
                        av = rows_v[e, pl.ds(v * D + k * 16, 16)]
                        accd = accd + av * bks[k]
                    dbuf[pl.ds(j * 16, 16)] = accd
                gidx = iota * 16
                dsum = plsc.load_gather(dbuf, [gidx])
                for j in range(1, 16):
                    dsum = dsum + plsc.load_gather(dbuf, [gidx + j])
                obuf[e, pl.ds(g * 16, 16)] = dsum
            return ecarry

        lax.fori_loop(0, CH, elem_body, 0)
        pltpu.sync_copy(obuf, out.at[pl.ds(eb, CH)])
        return carry

    lax.fori_loop(0, NCHUNK, chunk_body, 0)


@jax.jit
def _cosine(gemd, cid, prompts3):
    pn = pl.pallas_call(
        _pnorm_body,
        out_shape=jax.ShapeDtypeStruct((C, V, D), jnp.float32),
        grid=(C // 8,),
        in_specs=[pl.BlockSpec((8, V, D), lambda i: (i, 0, 0))],
        out_specs=pl.BlockSpec((8, V, D), lambda i: (i, 0, 0)),
    )(prompts3)
    gn = pl.pallas_call(
        _gnorm_body,
        out_shape=jax.ShapeDtypeStruct((B, D), jnp.float32),
        grid=(B // 512,),
        in_specs=[pl.BlockSpec((512, D), lambda i: (i, 0))],
        out_specs=pl.BlockSpec((512, D), lambda i: (i, 0)),
    )(gemd)
    ptab = pn.reshape(C, ROW)

    mesh = plsc.VectorSubcoreMesh(core_axis_name="c", subcore_axis_name="s")
    run = functools.partial(
        pl.kernel,
        mesh=mesh,
        out_type=jax.ShapeDtypeStruct((B, VPAD), jnp.float32),
        compiler_params=pltpu.CompilerParams(needs_layout_passes=False),
        scratch_types=[
            pltpu.VMEM((CH,), jnp.int32),
            pltpu.VMEM((CH, D), jnp.float32),
            pltpu.VMEM((CH, ROW), jnp.float32),
            pltpu.VMEM((256,), jnp.float32),
            pltpu.VMEM((CH, VPAD), jnp.float32),
            pltpu.SemaphoreType.DMA,
        ],
    )(_sc_body)
    return run(gn, cid, ptab)


def kernel(graph_emd, cluster_id, prompts):
    cid = cluster_id.astype(jnp.int32)
    out = _cosine(graph_emd, cid, prompts.reshape(C, V, D))
    return out[:, :V].reshape(B, 10, 5)


# speedup vs baseline: 1.1978x; 1.1978x over previous
import functools

import jax
import jax.numpy as jnp
from jax import lax
from jax.experimental import pallas as pl
from jax.experimental.pallas import tpu as pltpu
from jax.experimental.pallas import tpu_sc as plsc

B = 4096
C = 1000
T = 10
P = 5
V = T * P
D = 128
VPAD = 64
NW = 32
EPW = B // NW
CH = 8
NCHUNK = EPW // CH
NK = D // 16
EPS2 = 1e-16


def _rnorm_body(x_ref, o_ref):
    x = x_ref[...]
    n2 = jnp.sum(x * x, axis=-1)
    rn = lax.rsqrt(jnp.maximum(n2, EPS2)).reshape(8, V)
    o_ref[...] = jnp.concatenate(
        [rn, jnp.zeros((8, D - V), jnp.float32)], axis=1)


def _gnorm_body(x_ref, o_ref):
    x = x_ref[...]
    n2 = jnp.sum(x * x, axis=-1, keepdims=True)
    o_ref[...] = x * lax.rsqrt(jnp.maximum(n2, EPS2))


def _sc_body(gemd, cid, ptab, rn, out, idx_v, b_v, rows_v, rn_v, dbuf, obuf,
             sem, sem2):
    wid = lax.axis_index("s") * 2 + lax.axis_index("c")
    base = wid * EPW
    iota = lax.iota(jnp.int32, 16)

    def chunk_body(c, carry):
        eb = base + c * CH
        pltpu.sync_copy(cid.at[pl.ds(eb, CH)], idx_v)
        pltpu.sync_copy(gemd.at[pl.ds(eb, CH)], b_v)
        rcp = pltpu.async_copy(ptab.at[idx_v], rows_v, sem)
        ncp = pltpu.async_copy(rn.at[idx_v], rn_v, sem2)
        rcp.wait()
        ncp.wait()

        def elem_body(e, ecarry):
            bks = [b_v[e, pl.ds(k * 16, 16)] for k in range(NK)]
            for g in range(4):
                nj = 16 if g < 3 else V - 48
                accs = [rows_v[e, pl.ds((g * 16 + j) * D, 16)] * bks[0]
                        for j in range(nj)]
                for k in range(1, NK):
                    for j in range(nj):
                        av = rows_v[e, pl.ds((g * 16 + j) * D + k * 16, 16)]
                        accs[j] = accs[j] + av * bks[k]
                for j in range(nj):
                    dbuf[pl.ds(j * 16, 16)] = accs[j]
                gidx = iota * 16
                parts = [plsc.load_gather(dbuf, [gidx + j]) for j in range(16)]
                while len(parts) > 1:
                    parts = [parts[i] + parts[i + 1]
                             for i in range(0, len(parts), 2)]
                rna = rn_v[e, pl.ds(g * 16, 16)]
                obuf[e, pl.ds(g * 16, 16)] = parts[0] * rna
            return ecarry

        lax.fori_loop(0, CH, elem_body, 0)
        pltpu.sync_copy(obuf, out.at[pl.ds(eb, CH)])
        return carry

    lax.fori_loop(0, NCHUNK, chunk_body, 0)


@jax.jit
def _cosine(gemd, cid, prompts):
    rn = pl.pallas_call(
        _rnorm_body,
        out_shape=jax.ShapeDtypeStruct((C, D), jnp.float32),
        grid=(C // 8,),
        in_specs=[pl.BlockSpec((8, T, P, D), lambda i: (i, 0, 0, 0))],
        out_specs=pl.BlockSpec((8, D), lambda i: (i, 0)),
    )(prompts)
    gn = pl.pallas_call(
        _gnorm_body,
        out_shape=jax.ShapeDtypeStruct((B, D), jnp.float32),
        grid=(B // 512,),
        in_specs=[pl.BlockSpec((512, D), lambda i: (i, 0))],
        out_specs=pl.BlockSpec((512, D), lambda i: (i, 0)),
    )(gemd)

    mesh = plsc.VectorSubcoreMesh(core_axis_name="c", subcore_axis_name="s")
    run = functools.partial(
        pl.kernel,
        mesh=mesh,
        out_type=jax.ShapeDtypeStruct((B, VPAD), jnp.float32),
        compiler_params=pltpu.CompilerParams(needs_layout_passes=False),
        scratch_types=[
            pltpu.VMEM((CH,), jnp.int32),
            pltpu.VMEM((CH, D), jnp.float32),
            pltpu.VMEM((CH, T * P * D), jnp.float32),
            pltpu.VMEM((CH, D), jnp.float32),
            pltpu.VMEM((256,), jnp.float32),
            pltpu.VMEM((CH, VPAD), jnp.float32),
            pltpu.SemaphoreType.DMA,
            pltpu.SemaphoreType.DMA,
        ],
    )(_sc_body)
    return run(gn, cid, prompts.reshape(C, T * P * D), rn)


def kernel(graph_emd, cluster_id, prompts):
    cid = cluster_id.astype(jnp.int32)
    out = _cosine(graph_emd, cid, prompts)
    return out[:, :V].reshape(B, T, P)


# speedup vs baseline: 1.6685x; 1.3929x over previous
import functools

import jax
import jax.numpy as jnp
from jax import lax
from jax.experimental import pallas as pl
from jax.experimental.pallas import tpu as pltpu
from jax.experimental.pallas import tpu_sc as plsc

B = 4096
C = 1000
T = 10
P = 5
V = T * P
D = 128
ROW = V * D
VPAD = 64
NW = 32
EPW = B // NW
CH = 8
NCHUNK = EPW // CH
NK = D // 16
EPS2 = 1e-16


def _rsqrt16(x):
    i = plsc.bitcast(x, jnp.int32)
    i = jnp.int32(0x5F3759DF) - lax.shift_right_arithmetic(i, 1)
    y = plsc.bitcast(i, jnp.float32)
    for _ in range(3):
        y = y * (jnp.float32(1.5) - jnp.float32(0.5) * x * y * y)
    return y


def _sc_body(gemd, cid, ptab, out, idx_v, b_all, rows_v, dbuf, nbuf, tbuf,
             obuf, sem):
    wid = lax.axis_index("s") * 2 + lax.axis_index("c")
    base = wid * EPW
    iota = lax.iota(jnp.int32, 16)
    xor_masks = [iota ^ m for m in (8, 4, 2, 1)]

    pltpu.sync_copy(gemd.at[pl.ds(base, EPW)], b_all)

    def norm_body(r, carry):
        bks = [b_all[r, pl.ds(k * 16, 16)] for k in range(NK)]
        acc = bks[0] * bks[0]
        for k in range(1, NK):
            acc = acc + bks[k] * bks[k]
        for m in xor_masks:
            tbuf[pl.ds(0, 16)] = acc
            acc = acc + plsc.load_gather(tbuf, [m])
        rnb = _rsqrt16(jnp.maximum(acc, jnp.float32(EPS2)))
        for k in range(NK):
            b_all[r, pl.ds(k * 16, 16)] = bks[k] * rnb
        return carry

    lax.fori_loop(0, EPW, norm_body, 0)

    def chunk_body(c, carry):
        eb = base + c * CH
        pltpu.sync_copy(cid.at[pl.ds(eb, CH)], idx_v)
        pltpu.async_copy(ptab.at[idx_v], rows_v, sem).wait()

        def elem_body(e, ecarry):
            ce = c * CH + e
            bks = [b_all[ce, pl.ds(k * 16, 16)] for k in range(NK)]
            for g in range(4):
                nj = 16 if g < 3 else V - 48
                accd = []
                accn = []
                for j in range(nj):
                    av = rows_v[e, pl.ds((g * 16 + j) * D, 16)]
                    accd.append(av * bks[0])
                    accn.append(av * av)
                for k in range(1, NK):
                    for j in range(nj):
                        av = rows_v[e, pl.ds((g * 16 + j) * D + k * 16, 16)]
                        accd[j] = accd[j] + av * bks[k]
                        accn[j] = accn[j] + av * av
                for j in range(nj):
                    dbuf[pl.ds(j * 16, 16)] = accd[j]
                    nbuf[pl.ds(j * 16, 16)] = accn[j]
                gidx = iota * 16
                dparts = [plsc.load_gather(dbuf, [gidx + j])
                          for j in range(16)]
                nparts = [plsc.load_gather(nbuf, [gidx + j])
                          for j in range(16)]
                while len(dparts) > 1:
                    dparts = [dparts[i] + dparts[i + 1]
                              for i in range(0, len(dparts), 2)]
                    nparts = [nparts[i] + nparts[i + 1]
                              for i in range(0, len(nparts), 2)]
                rna = _rsqrt16(jnp.maximum(nparts[0], jnp.float32(EPS2)))
                obuf[e, pl.ds(g * 16, 16)] = dparts[0] * rna
            return ecarry

        lax.fori_loop(0, CH, elem_body, 0)
        pltpu.sync_copy(obuf, out.at[pl.ds(eb, CH)])
        return carry

    lax.fori_loop(0, NCHUNK, chunk_body, 0)


@jax.jit
def _cosine(gemd, cid, ptab):
    mesh = plsc.VectorSubcoreMesh(core_axis_name="c", subcore_axis_name="s")
    run = functools.partial(
        pl.kernel,
        mesh=mesh,
        out_type=jax.ShapeDtypeStruct((B, VPAD), jnp.float32),
        compiler_params=pltpu.CompilerParams(needs_layout_passes=False),
        scratch_types=[
            pltpu.VMEM((CH,), jnp.int32),
            pltpu.VMEM((EPW, D), jnp.float32),
            pltpu.VMEM((CH, ROW), jnp.float32),
            pltpu.VMEM((256,), jnp.float32),
            pltpu.VMEM((256,), jnp.float32),
            pltpu.VMEM((16,), jnp.float32),
            pltpu.VMEM((CH, VPAD), jnp.float32),
            pltpu.SemaphoreType.DMA,
        ],
    )(_sc_body)
    return run(gemd, cid, ptab)


def kernel(graph_emd, cluster_id, prompts):
    cid = cluster_id.astype(jnp.int32)
    out = _cosine(graph_emd, cid, prompts.reshape(C, ROW))
    return out[:, :V].reshape(B, T, P)
